# baseline (speedup 1.0000x reference)
import functools

import jax
import jax.numpy as jnp
from jax import lax
from jax.experimental import pallas as pl
from jax.experimental.pallas import tpu as pltpu
from jax.experimental.pallas import tpu_sc as plsc

_NC = 2
_NS = 16
_NW = _NC * _NS
_CHUNK = 128


def _make_sc_gather(n_out_rows, n_table_rows, d):
    n_chunks = n_out_rows // _CHUNK
    k_max = (n_chunks + _NW - 1) // _NW
    mesh = plsc.VectorSubcoreMesh(core_axis_name="c", subcore_axis_name="s")

    @functools.partial(
        pl.kernel,
        mesh=mesh,
        out_type=jax.ShapeDtypeStruct((n_out_rows, d), jnp.float32),
        scratch_types=[
            pltpu.VMEM((1, _CHUNK), jnp.int32),
            pltpu.VMEM((_CHUNK, d), jnp.float32),
            pltpu.SemaphoreType.DMA,
        ],
    )
    def gather(table_hbm, idx_hbm, out_hbm, idx_v, rows_v, sem):
        w = lax.axis_index("s") * _NC + lax.axis_index("c")

        def body(k, carry):
            c = w + k * _NW

            @pl.when(c < n_chunks)
            def _():
                pltpu.sync_copy(idx_hbm.at[pl.ds(c * _CHUNK, _CHUNK)], idx_v.at[0])
                pltpu.async_copy(table_hbm.at[idx_v.at[0]], rows_v, sem).wait()
                pltpu.sync_copy(rows_v, out_hbm.at[pl.ds(c * _CHUNK, _CHUNK)])

            return carry

        lax.fori_loop(0, k_max, body, 0)

    return gather


def _make_sc_scatter(n_rows, d):
    n_chunks = n_rows // _CHUNK
    k_max = (n_chunks + _NW - 1) // _NW
    mesh = plsc.VectorSubcoreMesh(core_axis_name="c", subcore_axis_name="s")

    @functools.partial(
        pl.kernel,
        mesh=mesh,
        out_type=jax.ShapeDtypeStruct((n_rows, d), jnp.float32),
        scratch_types=[
            pltpu.VMEM((1, _CHUNK), jnp.int32),
            pltpu.VMEM((_CHUNK, d), jnp.float32),
            pltpu.SemaphoreType.DMA,
        ],
    )
    def scatter(vals_hbm, idx_hbm, out_hbm, idx_v, rows_v, sem):
        w = lax.axis_index("s") * _NC + lax.axis_index("c")

        def body(k, carry):
            c = w + k * _NW

            @pl.when(c < n_chunks)
            def _():
                pltpu.sync_copy(idx_hbm.at[pl.ds(c * _CHUNK, _CHUNK)], idx_v.at[0])
                pltpu.sync_copy(vals_hbm.at[pl.ds(c * _CHUNK, _CHUNK)], rows_v)
                pltpu.async_copy(rows_v, out_hbm.at[idx_v.at[0]], sem).wait()

            return carry

        lax.fori_loop(0, k_max, body, 0)

    return scatter


def _make_tc_body(T, D):
    def body(xp_ref, w1a_ref, w1b_ref, w2_ref, b1_ref, b2_ref,
             u1a_ref, u1b_ref, u2_ref, ub1_ref, ub2_ref,
             g_ref, beta_ref, out_ref):
        t = pl.program_id(1)
        f32 = jnp.float32
        bf16 = jnp.bfloat16
        A = xp_ref[0, pl.ds(t * T, T + 2), :]
        Ab = A.astype(bf16)
        f_h = jnp.dot(Ab, w1a_ref[...], preferred_element_type=f32)
        g_h = jnp.dot(Ab, w1b_ref[...], preferred_element_type=f32)
        xc = A[1:T + 1]
        f = f_h[1:T + 1]
        b1 = b1_ref[0]
        yp = f + g_h[0:T] + b1
        yn = f + g_h[2:T + 2] + b1
        s = yp * jax.nn.sigmoid(yp) + yn * jax.nn.sigmoid(yn)
        msg = (jnp.dot(s.astype(bf16), w2_ref[...], preferred_element_type=f32)
               + 2.0 * b2_ref[0])
        u = (jnp.dot(Ab[1:T + 1], u1a_ref[...], preferred_element_type=f32)
             + jnp.dot(msg.astype(bf16), u1b_ref[...], preferred_element_type=f32)
             + ub1_ref[0])
        u = u * jax.nn.sigmoid(u)
        upd = (jnp.dot(u.astype(bf16), u2_ref[...], preferred_element_type=f32)
               + ub2_ref[0])
        r = xc + upd
        mu = jnp.mean(r, axis=-1, keepdims=True)
        var = jnp.mean((r - mu) ** 2, axis=-1, keepdims=True)
        out_ref[0] = (r - mu) * lax.rsqrt(var + 1e-5) * g_ref[0] + beta_ref[0]

    return body


def _tc_compute(xp, weights, B, N, D, T):
    NT = N // T
    P = xp.shape[1]
    (w1a, w1b, w2, b1, b2, u1a, u1b, u2, ub1, ub2, g, beta) = weights

    def wspec(shape):
        return pl.BlockSpec(shape, lambda b, t: (0,) * len(shape))

    return pl.pallas_call(
        _make_tc_body(T, D),
        grid=(B, NT),
        in_specs=[
            pl.BlockSpec((1, P, D), lambda b, t: (b, 0, 0)),
            wspec((D, D)), wspec((D, D)), wspec((D, D)),
            wspec((1, D)), wspec((1, D)),
            wspec((D, D)), wspec((D, D)), wspec((D, D)),
            wspec((1, D)), wspec((1, D)),
            wspec((1, D)), wspec((1, D)),
        ],
        out_specs=pl.BlockSpec((1, T, D), lambda b, t: (b, t, 0)),
        out_shape=jax.ShapeDtypeStruct((B, N, D), jnp.float32),
        compiler_params=pltpu.CompilerParams(
            dimension_semantics=("arbitrary", "arbitrary"),
        ),
    )(xp, w1a, w1b, w2, b1, b2, u1a, u1b, u2, ub1, ub2, g, beta)


def kernel(h, tour, msg_w1, msg_b1, msg_w2, msg_b2,
           upd_w1, upd_b1, upd_w2, upd_b2, ln_g, ln_b):
    B, N, D = h.shape
    T = 1000
    P = N + 16

    base = (jnp.arange(B, dtype=jnp.int32) * N)[:, None]
    ext = jnp.concatenate(
        [tour[:, -1:], tour, tour[:, :1],
         jnp.zeros((B, P - N - 2), jnp.int32)], axis=1) + base
    h_flat = h.reshape(B * N, D)

    xp_flat = _make_sc_gather(B * P, B * N, D)(h_flat, ext.reshape(-1))
    xp = xp_flat.reshape(B, P, D)

    bf16 = jnp.bfloat16
    weights = (
        msg_w1[:D].astype(bf16), msg_w1[D:].astype(bf16), msg_w2.astype(bf16),
        msg_b1.reshape(1, D), msg_b2.reshape(1, D),
        upd_w1[:D].astype(bf16), upd_w1[D:].astype(bf16), upd_w2.astype(bf16),
        upd_b1.reshape(1, D), upd_b2.reshape(1, D),
        ln_g.reshape(1, D), ln_b.reshape(1, D),
    )
    h_new_pos = _tc_compute(xp, weights, B, N, D, T)

    flat_tour = (tour + base).reshape(-1)
    h_new_flat = _make_sc_scatter(B * N, D)(h_new_pos.reshape(B * N, D), flat_tour)
    return h_new_flat.reshape(B, N, D)


# speedup vs baseline: 1.2293x; 1.2293x over previous
import functools

import jax
import jax.numpy as jnp
from jax import lax
from jax.experimental import pallas as pl
from jax.experimental.pallas import tpu as pltpu
from jax.experimental.pallas import tpu_sc as plsc

_NC = 2
_NS = 16
_NW = _NC * _NS
_CHUNK = 128


_NBUF = 3


def _make_sc_gather(n_chunks, n_table_rows, d):
    k0 = (n_chunks + _NW - 1) // _NW
    ks = -(-k0 // 8) * 8
    mesh = plsc.VectorSubcoreMesh(core_axis_name="c", subcore_axis_name="s")

    @functools.partial(
        pl.kernel,
        mesh=mesh,
        out_type=jax.ShapeDtypeStruct((n_chunks * _CHUNK, d), jnp.float32),
        scratch_types=[
            pltpu.VMEM((ks, _CHUNK), jnp.int32),
            pltpu.VMEM((_NBUF, _CHUNK, d), jnp.float32),
            pltpu.SemaphoreType.DMA,
        ],
    )
    def gather(table_hbm, idx_hbm, out_hbm, idx_v, rows_v, sem):
        w = lax.axis_index("s") * _NC + lax.axis_index("c")
        c0 = w * k0
        pltpu.sync_copy(idx_hbm.at[pl.ds(w * ks, ks)], idx_v)

        for jj in range(min(_NBUF - 1, k0)):
            @pl.when(c0 + jj < n_chunks)
            def _(jj=jj):
                pltpu.async_copy(table_hbm.at[idx_v.at[jj]], rows_v.at[jj], sem)

        def body(j, carry):
            c = c0 + j

            @pl.when(c < n_chunks)
            def _():
                ja = j + _NBUF - 1

                @pl.when((ja < k0) & (c0 + ja < n_chunks))
                def _():
                    pltpu.async_copy(
                        table_hbm.at[idx_v.at[ja]], rows_v.at[ja % _NBUF], sem)

                pltpu.make_async_copy(
                    table_hbm.at[idx_v.at[j]], rows_v.at[j % _NBUF], sem).wait()
                pltpu.sync_copy(rows_v.at[j % _NBUF],
                                out_hbm.at[pl.ds(c * _CHUNK, _CHUNK)])

            return carry

        lax.fori_loop(0, k0, body, 0)

    return gather


def _make_sc_scatter(n_chunks, d):
    k0 = (n_chunks + _NW - 1) // _NW
    ks = -(-k0 // 8) * 8
    mesh = plsc.VectorSubcoreMesh(core_axis_name="c", subcore_axis_name="s")

    @functools.partial(
        pl.kernel,
        mesh=mesh,
        out_type=jax.ShapeDtypeStruct((n_chunks * _CHUNK, d), jnp.float32),
        scratch_types=[
            pltpu.VMEM((ks, _CHUNK), jnp.int32),
            pltpu.VMEM((_NBUF, _CHUNK, d), jnp.float32),
            pltpu.SemaphoreType.DMA,
        ],
    )
    def scatter(vals_hbm, idx_hbm, out_hbm, idx_v, rows_v, sem):
        w = lax.axis_index("s") * _NC + lax.axis_index("c")
        c0 = w * k0
        pltpu.sync_copy(idx_hbm.at[pl.ds(w * ks, ks)], idx_v)

        for jj in range(min(_NBUF - 1, k0)):
            @pl.when(c0 + jj < n_chunks)
            def _(jj=jj):
                pltpu.async_copy(
                    vals_hbm.at[pl.ds((c0 + jj) * _CHUNK, _CHUNK)],
                    rows_v.at[jj], sem)

        def body(j, carry):
            c = c0 + j

            @pl.when(c < n_chunks)
            def _():
                ja = j + _NBUF - 1

                @pl.when((ja < k0) & (c0 + ja < n_chunks))
                def _():
                    pltpu.async_copy(
                        vals_hbm.at[pl.ds((c0 + ja) * _CHUNK, _CHUNK)],
                        rows_v.at[ja % _NBUF], sem)

                pltpu.make_async_copy(
                    vals_hbm.at[pl.ds(c * _CHUNK, _CHUNK)],
                    rows_v.at[j % _NBUF], sem).wait()
                pltpu.sync_copy(rows_v.at[j % _NBUF], out_hbm.at[idx_v.at[j]])

            return carry

        lax.fori_loop(0, k0, body, 0)

    return scatter


def _make_tc_body(T, D):
    def body(xp_ref, w1a_ref, w1b_ref, w2_ref, b1_ref, b2_ref,
             u1a_ref, u1b_ref, u2_ref, ub1_ref, ub2_ref,
             g_ref, beta_ref, out_ref):
        t = pl.program_id(1)
        f32 = jnp.float32
        bf16 = jnp.bfloat16
        A = xp_ref[0, pl.ds(t * T, T + 2), :]
        Ab = A.astype(bf16)
        f_h = jnp.dot(Ab, w1a_ref[...], preferred_element_type=f32)
        g_h = jnp.dot(Ab, w1b_ref[...], preferred_element_type=f32)
        xc = A[1:T + 1]
        f = f_h[1:T + 1]
        b1 = b1_ref[0]
        yp = f + g_h[0:T] + b1
        yn = f + g_h[2:T + 2] + b1
        s = yp * jax.nn.sigmoid(yp) + yn * jax.nn.sigmoid(yn)
        msg = (jnp.dot(s.astype(bf16), w2_ref[...], preferred_element_type=f32)
               + 2.0 * b2_ref[0])
        u = (jnp.dot(Ab[1:T + 1], u1a_ref[...], preferred_element_type=f32)
             + jnp.dot(msg.astype(bf16), u1b_ref[...], preferred_element_type=f32)
             + ub1_ref[0])
        u = u * jax.nn.sigmoid(u)
        upd = (jnp.dot(u.astype(bf16), u2_ref[...], preferred_element_type=f32)
               + ub2_ref[0])
        r = xc + upd
        mu = jnp.mean(r, axis=-1, keepdims=True)
        var = jnp.mean((r - mu) ** 2, axis=-1, keepdims=True)
        out_ref[0] = (r - mu) * lax.rsqrt(var + 1e-5) * g_ref[0] + beta_ref[0]

    return body


def _tc_compute(xp, weights, B, N, D, T):
    NT = N // T
    P = xp.shape[1]
    (w1a, w1b, w2, b1, b2, u1a, u1b, u2, ub1, ub2, g, beta) = weights

    def wspec(shape):
        return pl.BlockSpec(shape, lambda b, t: (0,) * len(shape))

    return pl.pallas_call(
        _make_tc_body(T, D),
        grid=(B, NT),
        in_specs=[
            pl.BlockSpec((1, P, D), lambda b, t: (b, 0, 0)),
            wspec((D, D)), wspec((D, D)), wspec((D, D)),
            wspec((1, D)), wspec((1, D)),
            wspec((D, D)), wspec((D, D)), wspec((D, D)),
            wspec((1, D)), wspec((1, D)),
            wspec((1, D)), wspec((1, D)),
        ],
        out_specs=pl.BlockSpec((1, T, D), lambda b, t: (b, t, 0)),
        out_shape=jax.ShapeDtypeStruct((B, N, D), jnp.float32),
        compiler_params=pltpu.CompilerParams(
            dimension_semantics=("arbitrary", "arbitrary"),
        ),
    )(xp, w1a, w1b, w2, b1, b2, u1a, u1b, u2, ub1, ub2, g, beta)


def kernel(h, tour, msg_w1, msg_b1, msg_w2, msg_b2,
           upd_w1, upd_b1, upd_w2, upd_b2, ln_g, ln_b):
    B, N, D = h.shape
    T = 1000
    P = N + 16

    base = (jnp.arange(B, dtype=jnp.int32) * N)[:, None]
    ext = jnp.concatenate(
        [tour[:, -1:], tour, tour[:, :1],
         jnp.zeros((B, P - N - 2), jnp.int32)], axis=1) + base

    def pad_idx(flat_idx, n_chunks):
        k0 = (n_chunks + _NW - 1) // _NW
        ks = -(-k0 // 8) * 8
        padded = (jnp.zeros((_NW * k0 * _CHUNK,), jnp.int32)
                  .at[:flat_idx.size].set(flat_idx).reshape(_NW, k0, _CHUNK))
        out = jnp.zeros((_NW, ks, _CHUNK), jnp.int32).at[:, :k0, :].set(padded)
        return out.reshape(_NW * ks, _CHUNK)

    h_flat = h.reshape(B * N, D)
    ncg = (B * P) // _CHUNK
    xp_flat = _make_sc_gather(ncg, B * N, D)(h_flat, pad_idx(ext.reshape(-1), ncg))
    xp = xp_flat.reshape(B, P, D)

    bf16 = jnp.bfloat16
    weights = (
        msg_w1[:D].astype(bf16), msg_w1[D:].astype(bf16), msg_w2.astype(bf16),
        msg_b1.reshape(1, D), msg_b2.reshape(1, D),
        upd_w1[:D].astype(bf16), upd_w1[D:].astype(bf16), upd_w2.astype(bf16),
        upd_b1.reshape(1, D), upd_b2.reshape(1, D),
        ln_g.reshape(1, D), ln_b.reshape(1, D),
    )
    h_new_pos = _tc_compute(xp, weights, B, N, D, T)

    ncs = (B * N) // _CHUNK
    h_new_flat = _make_sc_scatter(ncs, D)(
        h_new_pos.reshape(B * N, D), pad_idx((tour + base).reshape(-1), ncs))
    return h_new_flat.reshape(B, N, D)


# speedup vs baseline: 1.2686x; 1.0320x over previous
import functools

import jax
import jax.numpy as jnp
from jax import lax
from jax.experimental import pallas as pl
from jax.experimental.pallas import tpu as pltpu
from jax.experimental.pallas import tpu_sc as plsc

_NC = 2
_NS = 16
_NW = _NC * _NS
_CHUNK = 128
_NBUF = 3


def _idx_block(idx_hbm, idx_v, w, k0):
    c0 = w * k0
    a = (c0 // 8) * 8
    pltpu.sync_copy(idx_hbm.at[pl.ds(a, idx_v.shape[0])], idx_v)
    return c0, c0 - a


def _make_sc_gather(n_chunks, d):
    assert n_chunks % _NW == 0
    k0 = n_chunks // _NW
    kb = (k0 // 8 + 2) * 8
    mesh = plsc.VectorSubcoreMesh(core_axis_name="c", subcore_axis_name="s")

    @functools.partial(
        pl.kernel,
        mesh=mesh,
        out_type=jax.ShapeDtypeStruct((n_chunks * _CHUNK, d), jnp.float32),
        scratch_types=[
            pltpu.VMEM((kb, _CHUNK), jnp.int32),
            pltpu.VMEM((_NBUF, _CHUNK, d), jnp.float32),
            pltpu.SemaphoreType.DMA,
        ],
    )
    def gather(table_hbm, idx_hbm, out_hbm, idx_v, rows_v, sem):
        w = lax.axis_index("s") * _NC + lax.axis_index("c")
        c0, dlt = _idx_block(idx_hbm, idx_v, w, k0)

        for jj in range(_NBUF - 1):
            pltpu.async_copy(table_hbm.at[idx_v.at[dlt + jj]], rows_v.at[jj], sem)

        def body(j, carry):
            ja = j + _NBUF - 1

            @pl.when(ja < k0)
            def _():
                pltpu.async_copy(
                    table_hbm.at[idx_v.at[dlt + ja]], rows_v.at[ja % _NBUF], sem)

            pltpu.make_async_copy(
                table_hbm.at[idx_v.at[dlt + j]], rows_v.at[j % _NBUF], sem).wait()
            pltpu.sync_copy(rows_v.at[j % _NBUF],
                            out_hbm.at[pl.ds((c0 + j) * _CHUNK, _CHUNK)])
            return carry

        lax.fori_loop(0, k0, body, 0)

    return gather


def _make_sc_scatter(n_chunks, d):
    k0 = (n_chunks + _NW - 1) // _NW
    kb = (k0 // 8 + 2) * 8
    mesh = plsc.VectorSubcoreMesh(core_axis_name="c", subcore_axis_name="s")

    @functools.partial(
        pl.kernel,
        mesh=mesh,
        out_type=jax.ShapeDtypeStruct((n_chunks * _CHUNK, d), jnp.float32),
        scratch_types=[
            pltpu.VMEM((kb, _CHUNK), jnp.int32),
            pltpu.VMEM((_NBUF, _CHUNK, d), jnp.float32),
            pltpu.SemaphoreType.DMA,
        ],
    )
    def scatter(vals_hbm, idx_hbm, out_hbm, idx_v, rows_v, sem):
        w = lax.axis_index("s") * _NC + lax.axis_index("c")
        c0, dlt = _idx_block(idx_hbm, idx_v, w, k0)

        for jj in range(_NBUF - 1):
            @pl.when(c0 + jj < n_chunks)
            def _(jj=jj):
                pltpu.async_copy(
                    vals_hbm.at[pl.ds((c0 + jj) * _CHUNK, _CHUNK)],
                    rows_v.at[jj], sem)

        def body(j, carry):
            c = c0 + j

            @pl.when(c < n_chunks)
            def _():
                ja = j + _NBUF - 1

                @pl.when((ja < k0) & (c0 + ja < n_chunks))
                def _():
                    pltpu.async_copy(
                        vals_hbm.at[pl.ds((c0 + ja) * _CHUNK, _CHUNK)],
                        rows_v.at[ja % _NBUF], sem)

                pltpu.make_async_copy(
                    vals_hbm.at[pl.ds(c * _CHUNK, _CHUNK)],
                    rows_v.at[j % _NBUF], sem).wait()
                pltpu.sync_copy(rows_v.at[j % _NBUF], out_hbm.at[idx_v.at[dlt + j]])

            return carry

        lax.fori_loop(0, k0, body, 0)

    return scatter


def _make_tc_body(T, D):
    def body(xp_ref, w1a_ref, w1b_ref, w2_ref, b1_ref, b2_ref,
             u1a_ref, u1b_ref, u2_ref, ub1_ref, ub2_ref,
             g_ref, beta_ref, out_ref):
        t = pl.program_id(1)
        f32 = jnp.float32
        bf16 = jnp.bfloat16

        def silu(x):
            return x / (1.0 + jnp.exp(-x))

        A = xp_ref[0, pl.ds(t * T, T + 2), :]
        Ab = A.astype(bf16)
        f_h = jnp.dot(Ab, w1a_ref[...], preferred_element_type=f32)
        g_h = jnp.dot(Ab, w1b_ref[...], preferred_element_type=f32)
        xc = A[1:T + 1]
        f = f_h[1:T + 1]
        b1 = b1_ref[0]
        yp = f + g_h[0:T] + b1
        yn = f + g_h[2:T + 2] + b1
        s = silu(yp) + silu(yn)
        msg = (jnp.dot(s.astype(bf16), w2_ref[...], preferred_element_type=f32)
               + 2.0 * b2_ref[0])
        u = (jnp.dot(Ab[1:T + 1], u1a_ref[...], preferred_element_type=f32)
             + jnp.dot(msg.astype(bf16), u1b_ref[...], preferred_element_type=f32)
             + ub1_ref[0])
        u = silu(u)
        upd = (jnp.dot(u.astype(bf16), u2_ref[...], preferred_element_type=f32)
               + ub2_ref[0])
        r = xc + upd
        mu = jnp.mean(r, axis=-1, keepdims=True)
        var = jnp.mean((r - mu) ** 2, axis=-1, keepdims=True)
        out_ref[0] = (r - mu) * lax.rsqrt(var + 1e-5) * g_ref[0] + beta_ref[0]

    return body


def _tc_compute(xp, weights, B, N, D, T):
    NT = N // T
    P = xp.shape[1]
    (w1a, w1b, w2, b1, b2, u1a, u1b, u2, ub1, ub2, g, beta) = weights

    def wspec(shape):
        return pl.BlockSpec(shape, lambda b, t: (0,) * len(shape))

    return pl.pallas_call(
        _make_tc_body(T, D),
        grid=(B, NT),
        in_specs=[
            pl.BlockSpec((1, P, D), lambda b, t: (b, 0, 0)),
            wspec((D, D)), wspec((D, D)), wspec((D, D)),
            wspec((1, D)), wspec((1, D)),
            wspec((D, D)), wspec((D, D)), wspec((D, D)),
            wspec((1, D)), wspec((1, D)),
            wspec((1, D)), wspec((1, D)),
        ],
        out_specs=pl.BlockSpec((1, T, D), lambda b, t: (b, t, 0)),
        out_shape=jax.ShapeDtypeStruct((B, N, D), jnp.float32),
        compiler_params=pltpu.CompilerParams(
            dimension_semantics=("arbitrary", "arbitrary"),
        ),
    )(xp, w1a, w1b, w2, b1, b2, u1a, u1b, u2, ub1, ub2, g, beta)


def kernel(h, tour, msg_w1, msg_b1, msg_w2, msg_b2,
           upd_w1, upd_b1, upd_w2, upd_b2, ln_g, ln_b):
    B, N, D = h.shape
    T = 2000
    P = 10240

    base = (jnp.arange(B, dtype=jnp.int32) * N)[:, None]
    ext = jnp.concatenate(
        [tour[:, -1:], tour, tour[:, :1],
         jnp.zeros((B, P - N - 2), jnp.int32)], axis=1) + base
    h_flat = h.reshape(B * N, D)

    ncg = (B * P) // _CHUNK
    gidx = jnp.concatenate([ext.reshape(-1), jnp.zeros((8 * _CHUNK,), jnp.int32)])
    xp_flat = _make_sc_gather(ncg, D)(h_flat, gidx.reshape(ncg + 8, _CHUNK))
    xp = xp_flat.reshape(B, P, D)

    bf16 = jnp.bfloat16
    weights = (
        msg_w1[:D].astype(bf16), msg_w1[D:].astype(bf16), msg_w2.astype(bf16),
        msg_b1.reshape(1, D), msg_b2.reshape(1, D),
        upd_w1[:D].astype(bf16), upd_w1[D:].astype(bf16), upd_w2.astype(bf16),
        upd_b1.reshape(1, D), upd_b2.reshape(1, D),
        ln_g.reshape(1, D), ln_b.reshape(1, D),
    )
    h_new_pos = _tc_compute(xp, weights, B, N, D, T)

    ncs = (B * N) // _CHUNK
    flat_tour = (tour + base).reshape(-1)
    k0 = (ncs + _NW - 1) // _NW
    pad_rows = _NW * k0 + 8 - ncs
    sidx = jnp.concatenate([flat_tour, flat_tour[:pad_rows * _CHUNK]])
    h_new_flat = _make_sc_scatter(ncs, D)(
        h_new_pos.reshape(B * N, D), sidx.reshape(ncs + pad_rows, _CHUNK))
    return h_new_flat.reshape(B, N, D)


# speedup vs baseline: 1.3029x; 1.0270x over previous
import functools

import jax
import jax.numpy as jnp
from jax import lax
from jax.experimental import pallas as pl
from jax.experimental.pallas import tpu as pltpu
from jax.experimental.pallas import tpu_sc as plsc

_NC = 2
_NS = 16
_NW = _NC * _NS
_CHUNK = 128
_NBUF = 3


def _idx_block(idx_hbm, idx_v, w, k0):
    c0 = w * k0
    a = (c0 // 8) * 8
    pltpu.sync_copy(idx_hbm.at[pl.ds(a, idx_v.shape[0])], idx_v)
    return c0, c0 - a


def _make_sc_gather(n_chunks, d):
    assert n_chunks % _NW == 0
    k0 = n_chunks // _NW
    kb = (k0 // 8 + 2) * 8
    mesh = plsc.VectorSubcoreMesh(core_axis_name="c", subcore_axis_name="s")

    @functools.partial(
        pl.kernel,
        mesh=mesh,
        out_type=jax.ShapeDtypeStruct((n_chunks * _CHUNK, d), jnp.float32),
        scratch_types=[
            pltpu.VMEM((kb, _CHUNK), jnp.int32),
            pltpu.VMEM((_NBUF, _CHUNK, d), jnp.float32),
            pltpu.SemaphoreType.DMA,
        ],
    )
    def gather(table_hbm, idx_hbm, out_hbm, idx_v, rows_v, sem):
        w = lax.axis_index("s") * _NC + lax.axis_index("c")
        c0, dlt = _idx_block(idx_hbm, idx_v, w, k0)

        for jj in range(_NBUF - 1):
            pltpu.async_copy(table_hbm.at[idx_v.at[dlt + jj]], rows_v.at[jj], sem)

        def body(j, carry):
            ja = j + _NBUF - 1

            @pl.when(ja < k0)
            def _():
                pltpu.async_copy(
                    table_hbm.at[idx_v.at[dlt + ja]], rows_v.at[ja % _NBUF], sem)

            pltpu.make_async_copy(
                table_hbm.at[idx_v.at[dlt + j]], rows_v.at[j % _NBUF], sem).wait()
            pltpu.sync_copy(rows_v.at[j % _NBUF],
                            out_hbm.at[pl.ds((c0 + j) * _CHUNK, _CHUNK)])
            return carry

        lax.fori_loop(0, k0, body, 0)

    return gather


def _make_sc_scatter(n_chunks, d):
    k0 = (n_chunks + _NW - 1) // _NW
    kb = (k0 // 8 + 2) * 8
    mesh = plsc.VectorSubcoreMesh(core_axis_name="c", subcore_axis_name="s")

    @functools.partial(
        pl.kernel,
        mesh=mesh,
        out_type=jax.ShapeDtypeStruct((n_chunks * _CHUNK, d), jnp.float32),
        scratch_types=[
            pltpu.VMEM((kb, _CHUNK), jnp.int32),
            pltpu.VMEM((_NBUF, _CHUNK, d), jnp.float32),
            pltpu.SemaphoreType.DMA,
        ],
    )
    def scatter(vals_hbm, idx_hbm, out_hbm, idx_v, rows_v, sem):
        w = lax.axis_index("s") * _NC + lax.axis_index("c")
        c0, dlt = _idx_block(idx_hbm, idx_v, w, k0)

        for jj in range(_NBUF - 1):
            @pl.when(c0 + jj < n_chunks)
            def _(jj=jj):
                pltpu.async_copy(
                    vals_hbm.at[pl.ds((c0 + jj) * _CHUNK, _CHUNK)],
                    rows_v.at[jj], sem)

        def body(j, carry):
            c = c0 + j

            @pl.when(c < n_chunks)
            def _():
                ja = j + _NBUF - 1

                @pl.when((ja < k0) & (c0 + ja < n_chunks))
                def _():
                    pltpu.async_copy(
                        vals_hbm.at[pl.ds((c0 + ja) * _CHUNK, _CHUNK)],
                        rows_v.at[ja % _NBUF], sem)

                pltpu.make_async_copy(
                    vals_hbm.at[pl.ds(c * _CHUNK, _CHUNK)],
                    rows_v.at[j % _NBUF], sem).wait()
                pltpu.sync_copy(rows_v.at[j % _NBUF], out_hbm.at[idx_v.at[dlt + j]])

            return carry

        lax.fori_loop(0, k0, body, 0)

    return scatter


def _make_tc_body(T, D):
    def body(xp_ref, w1a_ref, w1b_ref, w2_ref, b1_ref, b2_ref,
             u1a_ref, u1b_ref, u2_ref, ub1_ref, ub2_ref,
             g_ref, beta_ref, out_ref):
        t = pl.program_id(1)
        f32 = jnp.float32
        bf16 = jnp.bfloat16

        def silu(x):
            return x / (1.0 + jnp.exp2(x * -1.4426950408889634))

        A = xp_ref[0, pl.ds(t * T, T + 2), :]
        xc = A[1:T + 1]
        xcb = xc.astype(bf16)
        g_h = jnp.dot(A.astype(bf16), w1b_ref[...], preferred_element_type=f32)
        fb = (jnp.dot(xcb, w1a_ref[...], preferred_element_type=f32)
              + b1_ref[0])
        yp = fb + g_h[0:T]
        yn = fb + g_h[2:T + 2]
        s = silu(yp) + silu(yn)
        msg = (jnp.dot(s.astype(bf16), w2_ref[...], preferred_element_type=f32)
               + 2.0 * b2_ref[0])
        u = (jnp.dot(xcb, u1a_ref[...], preferred_element_type=f32)
             + jnp.dot(msg.astype(bf16), u1b_ref[...], preferred_element_type=f32)
             + ub1_ref[0])
        u = silu(u)
        upd = (jnp.dot(u.astype(bf16), u2_ref[...], preferred_element_type=f32)
               + ub2_ref[0])
        r = xc + upd
        mu = jnp.mean(r, axis=-1, keepdims=True)
        var = jnp.mean((r - mu) ** 2, axis=-1, keepdims=True)
        out_ref[0] = (r - mu) * lax.rsqrt(var + 1e-5) * g_ref[0] + beta_ref[0]

    return body


def _tc_compute(xp, weights, B, N, D, T):
    NT = N // T
    P = xp.shape[1]
    (w1a, w1b, w2, b1, b2, u1a, u1b, u2, ub1, ub2, g, beta) = weights

    def wspec(shape):
        return pl.BlockSpec(shape, lambda b, t: (0,) * len(shape))

    return pl.pallas_call(
        _make_tc_body(T, D),
        grid=(B, NT),
        in_specs=[
            pl.BlockSpec((1, P, D), lambda b, t: (b, 0, 0)),
            wspec((D, D)), wspec((D, D)), wspec((D, D)),
            wspec((1, D)), wspec((1, D)),
            wspec((D, D)), wspec((D, D)), wspec((D, D)),
            wspec((1, D)), wspec((1, D)),
            wspec((1, D)), wspec((1, D)),
        ],
        out_specs=pl.BlockSpec((1, T, D), lambda b, t: (b, t, 0)),
        out_shape=jax.ShapeDtypeStruct((B, N, D), jnp.float32),
        compiler_params=pltpu.CompilerParams(
            dimension_semantics=("arbitrary", "arbitrary"),
        ),
    )(xp, w1a, w1b, w2, b1, b2, u1a, u1b, u2, ub1, ub2, g, beta)


def kernel(h, tour, msg_w1, msg_b1, msg_w2, msg_b2,
           upd_w1, upd_b1, upd_w2, upd_b2, ln_g, ln_b):
    B, N, D = h.shape
    T = 2000
    P = 10240

    base = (jnp.arange(B, dtype=jnp.int32) * N)[:, None]
    ext = jnp.concatenate(
        [tour[:, -1:], tour, tour[:, :1],
         jnp.zeros((B, P - N - 2), jnp.int32)], axis=1) + base
    h_flat = h.reshape(B * N, D)

    ncg = (B * P) // _CHUNK
    gidx = jnp.concatenate([ext.reshape(-1), jnp.zeros((8 * _CHUNK,), jnp.int32)])
    xp_flat = _make_sc_gather(ncg, D)(h_flat, gidx.reshape(ncg + 8, _CHUNK))
    xp = xp_flat.reshape(B, P, D)

    bf16 = jnp.bfloat16
    weights = (
        msg_w1[:D].astype(bf16), msg_w1[D:].astype(bf16), msg_w2.astype(bf16),
        msg_b1.reshape(1, D), msg_b2.reshape(1, D),
        upd_w1[:D].astype(bf16), upd_w1[D:].astype(bf16), upd_w2.astype(bf16),
        upd_b1.reshape(1, D), upd_b2.reshape(1, D),
        ln_g.reshape(1, D), ln_b.reshape(1, D),
    )
    h_new_pos = _tc_compute(xp, weights, B, N, D, T)

    ncs = (B * N) // _CHUNK
    flat_tour = (tour + base).reshape(-1)
    k0 = (ncs + _NW - 1) // _NW
    pad_rows = _NW * k0 + 8 - ncs
    sidx = jnp.concatenate([flat_tour, flat_tour[:pad_rows * _CHUNK]])
    h_new_flat = _make_sc_scatter(ncs, D)(
        h_new_pos.reshape(B * N, D), sidx.reshape(ncs + pad_rows, _CHUNK))
    return h_new_flat.reshape(B, N, D)


# speedup vs baseline: 1.3057x; 1.0021x over previous
import functools

import jax
import jax.numpy as jnp
from jax import lax
from jax.experimental import pallas as pl
from jax.experimental.pallas import tpu as pltpu
from jax.experimental.pallas import tpu_sc as plsc

_NC = 2
_NS = 16
_NW = _NC * _NS
_CHUNK = 128
_NBUF = 3


def _idx_block(idx_hbm, idx_v, w, k0):
    c0 = w * k0
    a = (c0 // 8) * 8
    pltpu.sync_copy(idx_hbm.at[pl.ds(a, idx_v.shape[0])], idx_v)
    return c0, c0 - a


def _make_sc_gather(n_chunks, d):
    assert n_chunks % _NW == 0
    k0 = n_chunks // _NW
    kb = (k0 // 8 + 2) * 8
    mesh = plsc.VectorSubcoreMesh(core_axis_name="c", subcore_axis_name="s")

    @functools.partial(
        pl.kernel,
        mesh=mesh,
        out_type=jax.ShapeDtypeStruct((n_chunks * _CHUNK, d), jnp.float32),
        scratch_types=[
            pltpu.VMEM((kb, _CHUNK), jnp.int32),
            pltpu.VMEM((_NBUF, _CHUNK, d), jnp.float32),
            pltpu.SemaphoreType.DMA,
        ],
    )
    def gather(table_hbm, idx_hbm, out_hbm, idx_v, rows_v, sem):
        w = lax.axis_index("s") * _NC + lax.axis_index("c")
        c0, dlt = _idx_block(idx_hbm, idx_v, w, k0)

        for jj in range(_NBUF - 1):
            pltpu.async_copy(table_hbm.at[idx_v.at[dlt + jj]], rows_v.at[jj], sem)

        def body(j, carry):
            ja = j + _NBUF - 1

            @pl.when(ja < k0)
            def _():
                pltpu.async_copy(
                    table_hbm.at[idx_v.at[dlt + ja]], rows_v.at[ja % _NBUF], sem)

            pltpu.make_async_copy(
                table_hbm.at[idx_v.at[dlt + j]], rows_v.at[j % _NBUF], sem).wait()
            pltpu.sync_copy(rows_v.at[j % _NBUF],
                            out_hbm.at[pl.ds((c0 + j) * _CHUNK, _CHUNK)])
            return carry

        lax.fori_loop(0, k0, body, 0)

    return gather


def _make_sc_scatter2(n_chunks, d, hr):
    k0 = (n_chunks + _NW - 1) // _NW
    kb = (k0 // 8 + 2) * 8
    cb = hr // _CHUNK
    la = hr - cb * _CHUNK
    assert 0 < la < _CHUNK and la % 8 == 0
    wb, jb = cb // k0, cb % k0
    mesh = plsc.VectorSubcoreMesh(core_axis_name="c", subcore_axis_name="s")

    @functools.partial(
        pl.kernel,
        mesh=mesh,
        out_type=jax.ShapeDtypeStruct((n_chunks * _CHUNK, d), jnp.float32),
        scratch_types=[
            pltpu.VMEM((kb, _CHUNK), jnp.int32),
            pltpu.VMEM((_NBUF, _CHUNK, d), jnp.float32),
            pltpu.SemaphoreType.DMA,
        ],
    )
    def scatter(vals_a, vals_b, idx_hbm, out_hbm, idx_v, rows_v, sem):
        w = lax.axis_index("s") * _NC + lax.axis_index("c")
        c0, dlt = _idx_block(idx_hbm, idx_v, w, k0)

        def issue(cc, buf, pred):
            @pl.when((cc < cb) & pred)
            def _():
                pltpu.async_copy(
                    vals_a.at[pl.ds(cc * _CHUNK, _CHUNK)], rows_v.at[buf], sem)

            @pl.when((cc > cb) & (cc < n_chunks) & pred)
            def _():
                pltpu.async_copy(
                    vals_b.at[pl.ds(cc * _CHUNK - hr, _CHUNK)],
                    rows_v.at[buf], sem)

        for jj in range(_NBUF - 1):
            issue(c0 + jj, jj, jj < k0)

        def body(j, carry):
            c = c0 + j
            ja = j + _NBUF - 1
            issue(c0 + ja, ja % _NBUF, ja < k0)

            @pl.when((c < n_chunks) & (c != cb))
            def _():
                pltpu.make_async_copy(
                    vals_a.at[pl.ds(0, _CHUNK)], rows_v.at[j % _NBUF], sem).wait()
                pltpu.sync_copy(rows_v.at[j % _NBUF], out_hbm.at[idx_v.at[dlt + j]])

            return carry

        lax.fori_loop(0, k0, body, 0)

        @pl.when(w == wb)
        def _():
            pltpu.sync_copy(vals_a.at[pl.ds(cb * _CHUNK, la)],
                            rows_v.at[0, pl.ds(0, la)])
            pltpu.sync_copy(vals_b.at[pl.ds(0, _CHUNK - la)],
                            rows_v.at[0, pl.ds(la, _CHUNK - la)])
            pltpu.sync_copy(rows_v.at[0], out_hbm.at[idx_v.at[dlt + jb]])

    return scatter


def _make_tc_body(T, D):
    def body(xp_ref, w1a_ref, w1b_ref, w2_ref, b1_ref, b2_ref,
             u1a_ref, u1b_ref, u2_ref, ub1_ref, ub2_ref,
             g_ref, beta_ref, out_ref):
        t = pl.program_id(1)
        f32 = jnp.float32
        bf16 = jnp.bfloat16

        def silu(x):
            return x / (1.0 + jnp.exp2(x * -1.4426950408889634))

        A = xp_ref[0, pl.ds(t * T, T + 2), :]
        xc = A[1:T + 1]
        xcb = xc.astype(bf16)
        g_h = jnp.dot(A.astype(bf16), w1b_ref[...], preferred_element_type=f32)
        fb = (jnp.dot(xcb, w1a_ref[...], preferred_element_type=f32)
              + b1_ref[0])
        yp = fb + g_h[0:T]
        yn = fb + g_h[2:T + 2]
        s = silu(yp) + silu(yn)
        msg = (jnp.dot(s.astype(bf16), w2_ref[...], preferred_element_type=f32)
               + 2.0 * b2_ref[0])
        u = (jnp.dot(xcb, u1a_ref[...], preferred_element_type=f32)
             + jnp.dot(msg.astype(bf16), u1b_ref[...], preferred_element_type=f32)
             + ub1_ref[0])
        u = silu(u)
        upd = (jnp.dot(u.astype(bf16), u2_ref[...], preferred_element_type=f32)
               + ub2_ref[0])
        r = xc + upd
        mu = jnp.mean(r, axis=-1, keepdims=True)
        var = jnp.mean((r - mu) ** 2, axis=-1, keepdims=True)
        out_ref[0] = (r - mu) * lax.rsqrt(var + 1e-5) * g_ref[0] + beta_ref[0]

    return body


def _tc_compute(xp, weights, B, N, D, T):
    NT = N // T
    P = xp.shape[1]
    (w1a, w1b, w2, b1, b2, u1a, u1b, u2, ub1, ub2, g, beta) = weights

    def wspec(shape):
        return pl.BlockSpec(shape, lambda b, t: (0,) * len(shape))

    return pl.pallas_call(
        _make_tc_body(T, D),
        grid=(B, NT),
        in_specs=[
            pl.BlockSpec((1, P, D), lambda b, t: (b, 0, 0)),
            wspec((D, D)), wspec((D, D)), wspec((D, D)),
            wspec((1, D)), wspec((1, D)),
            wspec((D, D)), wspec((D, D)), wspec((D, D)),
            wspec((1, D)), wspec((1, D)),
            wspec((1, D)), wspec((1, D)),
        ],
        out_specs=pl.BlockSpec((1, T, D), lambda b, t: (b, t, 0)),
        out_shape=jax.ShapeDtypeStruct((B, N, D), jnp.float32),
        compiler_params=pltpu.CompilerParams(
            dimension_semantics=("arbitrary", "arbitrary"),
        ),
    )(xp, w1a, w1b, w2, b1, b2, u1a, u1b, u2, ub1, ub2, g, beta)


def kernel(h, tour, msg_w1, msg_b1, msg_w2, msg_b2,
           upd_w1, upd_b1, upd_w2, upd_b2, ln_g, ln_b):
    B, N, D = h.shape
    T = 2000
    P = 10240

    H = B // 2
    base = (jnp.arange(B, dtype=jnp.int32) * N)[:, None]
    ext = jnp.concatenate(
        [tour[:, -1:], tour, tour[:, :1],
         jnp.zeros((B, P - N - 2), jnp.int32)], axis=1) + base
    h_flat = h.reshape(B * N, D)

    nch = (H * P) // _CHUNK
    gather = _make_sc_gather(nch, D)

    def half_idx(e):
        g = jnp.concatenate([e.reshape(-1), jnp.zeros((8 * _CHUNK,), jnp.int32)])
        return g.reshape(nch + 8, _CHUNK)

    xp_a = gather(h_flat, half_idx(ext[:H])).reshape(H, P, D)
    xp_b = gather(h_flat, half_idx(ext[H:])).reshape(H, P, D)

    bf16 = jnp.bfloat16
    weights = (
        msg_w1[:D].astype(bf16), msg_w1[D:].astype(bf16), msg_w2.astype(bf16),
        msg_b1.reshape(1, D), msg_b2.reshape(1, D),
        upd_w1[:D].astype(bf16), upd_w1[D:].astype(bf16), upd_w2.astype(bf16),
        upd_b1.reshape(1, D), upd_b2.reshape(1, D),
        ln_g.reshape(1, D), ln_b.reshape(1, D),
    )
    hn_a = _tc_compute(xp_a, weights, H, N, D, T).reshape(H * N, D)
    hn_b = _tc_compute(xp_b, weights, H, N, D, T).reshape(H * N, D)

    ncs = (B * N) // _CHUNK
    flat_tour = (tour + base).reshape(-1)
    k0 = (ncs + _NW - 1) // _NW
    pad_rows = _NW * k0 + 8 - ncs
    sidx = jnp.concatenate([flat_tour, flat_tour[:pad_rows * _CHUNK]])
    h_new_flat = _make_sc_scatter2(ncs, D, H * N)(
        hn_a, hn_b, sidx.reshape(ncs + pad_rows, _CHUNK))
    return h_new_flat.reshape(B, N, D)


# speedup vs baseline: 1.3097x; 1.0031x over previous
import functools

import jax
import jax.numpy as jnp
from jax import lax
from jax.experimental import pallas as pl
from jax.experimental.pallas import tpu as pltpu
from jax.experimental.pallas import tpu_sc as plsc

_NC = 2
_NS = 16
_NW = _NC * _NS
_CHUNK = 128
_NBUF = 4


def _idx_block(idx_hbm, idx_v, w, k0):
    c0 = w * k0
    a = (c0 // 8) * 8
    pltpu.sync_copy(idx_hbm.at[pl.ds(a, idx_v.shape[0])], idx_v)
    return c0, c0 - a


def _make_sc_gather(n_chunks, d):
    assert n_chunks % _NW == 0
    k0 = n_chunks // _NW
    kb = (k0 // 8 + 2) * 8
    mesh = plsc.VectorSubcoreMesh(core_axis_name="c", subcore_axis_name="s")

    @functools.partial(
        pl.kernel,
        mesh=mesh,
        out_type=jax.ShapeDtypeStruct((n_chunks * _CHUNK, d), jnp.float32),
        scratch_types=[
            pltpu.VMEM((kb, _CHUNK), jnp.int32),
            pltpu.VMEM((_NBUF, _CHUNK, d), jnp.float32),
            pltpu.SemaphoreType.DMA,
        ],
    )
    def gather(table_hbm, idx_hbm, out_hbm, idx_v, rows_v, sem):
        w = lax.axis_index("s") * _NC + lax.axis_index("c")
        c0, dlt = _idx_block(idx_hbm, idx_v, w, k0)

        for jj in range(_NBUF - 1):
            pltpu.async_copy(table_hbm.at[idx_v.at[dlt + jj]], rows_v.at[jj], sem)

        def body(j, carry):
            ja = j + _NBUF - 1

            @pl.when(ja < k0)
            def _():
                pltpu.async_copy(
                    table_hbm.at[idx_v.at[dlt + ja]], rows_v.at[ja % _NBUF], sem)

            pltpu.make_async_copy(
                table_hbm.at[idx_v.at[dlt + j]], rows_v.at[j % _NBUF], sem).wait()
            pltpu.sync_copy(rows_v.at[j % _NBUF],
                            out_hbm.at[pl.ds((c0 + j) * _CHUNK, _CHUNK)])
            return carry

        lax.fori_loop(0, k0, body, 0)

    return gather


def _make_sc_scatter2(n_chunks, d, hr):
    k0 = (n_chunks + _NW - 1) // _NW
    kb = (k0 // 8 + 2) * 8
    cb = hr // _CHUNK
    la = hr - cb * _CHUNK
    assert 0 < la < _CHUNK and la % 8 == 0
    wb, jb = cb // k0, cb % k0
    mesh = plsc.VectorSubcoreMesh(core_axis_name="c", subcore_axis_name="s")

    @functools.partial(
        pl.kernel,
        mesh=mesh,
        out_type=jax.ShapeDtypeStruct((n_chunks * _CHUNK, d), jnp.float32),
        scratch_types=[
            pltpu.VMEM((kb, _CHUNK), jnp.int32),
            pltpu.VMEM((_NBUF, _CHUNK, d), jnp.float32),
            pltpu.SemaphoreType.DMA,
        ],
    )
    def scatter(vals_a, vals_b, idx_hbm, out_hbm, idx_v, rows_v, sem):
        w = lax.axis_index("s") * _NC + lax.axis_index("c")
        c0, dlt = _idx_block(idx_hbm, idx_v, w, k0)

        def issue(cc, buf, pred):
            @pl.when((cc < cb) & pred)
            def _():
                pltpu.async_copy(
                    vals_a.at[pl.ds(cc * _CHUNK, _CHUNK)], rows_v.at[buf], sem)

            @pl.when((cc > cb) & (cc < n_chunks) & pred)
            def _():
                pltpu.async_copy(
                    vals_b.at[pl.ds(cc * _CHUNK - hr, _CHUNK)],
                    rows_v.at[buf], sem)

        for jj in range(_NBUF - 1):
            issue(c0 + jj, jj, jj < k0)

        def body(j, carry):
            c = c0 + j
            ja = j + _NBUF - 1
            issue(c0 + ja, ja % _NBUF, ja < k0)

            @pl.when((c < n_chunks) & (c != cb))
            def _():
                pltpu.make_async_copy(
                    vals_a.at[pl.ds(0, _CHUNK)], rows_v.at[j % _NBUF], sem).wait()
                pltpu.sync_copy(rows_v.at[j % _NBUF], out_hbm.at[idx_v.at[dlt + j]])

            return carry

        lax.fori_loop(0, k0, body, 0)

        @pl.when(w == wb)
        def _():
            pltpu.sync_copy(vals_a.at[pl.ds(cb * _CHUNK, la)],
                            rows_v.at[0, pl.ds(0, la)])
            pltpu.sync_copy(vals_b.at[pl.ds(0, _CHUNK - la)],
                            rows_v.at[0, pl.ds(la, _CHUNK - la)])
            pltpu.sync_copy(rows_v.at[0], out_hbm.at[idx_v.at[dlt + jb]])

    return scatter


def _make_tc_body(T, D):
    def body(xp_ref, w1a_ref, w1b_ref, w2_ref, b1_ref, b2_ref,
             u1a_ref, u1b_ref, u2_ref, ub1_ref, ub2_ref,
             g_ref, beta_ref, out_ref):
        t = pl.program_id(1)
        f32 = jnp.float32
        bf16 = jnp.bfloat16

        def silu(x):
            return x / (1.0 + jnp.exp2(x * -1.4426950408889634))

        A = xp_ref[0, pl.ds(t * T, T + 2), :]
        xc = A[1:T + 1]
        xcb = xc.astype(bf16)
        g_h = jnp.dot(A.astype(bf16), w1b_ref[...], preferred_element_type=f32)
        fb = (jnp.dot(xcb, w1a_ref[...], preferred_element_type=f32)
              + b1_ref[0])
        yp = fb + g_h[0:T]
        yn = fb + g_h[2:T + 2]
        s = silu(yp) + silu(yn)
        msg = (jnp.dot(s.astype(bf16), w2_ref[...], preferred_element_type=f32)
               + 2.0 * b2_ref[0])
        u = (jnp.dot(xcb, u1a_ref[...], preferred_element_type=f32)
             + jnp.dot(msg.astype(bf16), u1b_ref[...], preferred_element_type=f32)
             + ub1_ref[0])
        u = silu(u)
        upd = (jnp.dot(u.astype(bf16), u2_ref[...], preferred_element_type=f32)
               + ub2_ref[0])
        r = xc + upd
        mu = jnp.mean(r, axis=-1, keepdims=True)
        var = jnp.mean((r - mu) ** 2, axis=-1, keepdims=True)
        out_ref[0] = (r - mu) * lax.rsqrt(var + 1e-5) * g_ref[0] + beta_ref[0]

    return body


def _tc_compute(xp, weights, B, N, D, T):
    NT = N // T
    P = xp.shape[1]
    (w1a, w1b, w2, b1, b2, u1a, u1b, u2, ub1, ub2, g, beta) = weights

    def wspec(shape):
        return pl.BlockSpec(shape, lambda b, t: (0,) * len(shape))

    return pl.pallas_call(
        _make_tc_body(T, D),
        grid=(B, NT),
        in_specs=[
            pl.BlockSpec((1, P, D), lambda b, t: (b, 0, 0)),
            wspec((D, D)), wspec((D, D)), wspec((D, D)),
            wspec((1, D)), wspec((1, D)),
            wspec((D, D)), wspec((D, D)), wspec((D, D)),
            wspec((1, D)), wspec((1, D)),
            wspec((1, D)), wspec((1, D)),
        ],
        out_specs=pl.BlockSpec((1, T, D), lambda b, t: (b, t, 0)),
        out_shape=jax.ShapeDtypeStruct((B, N, D), jnp.float32),
        compiler_params=pltpu.CompilerParams(
            dimension_semantics=("arbitrary", "arbitrary"),
        ),
    )(xp, w1a, w1b, w2, b1, b2, u1a, u1b, u2, ub1, ub2, g, beta)


def kernel(h, tour, msg_w1, msg_b1, msg_w2, msg_b2,
           upd_w1, upd_b1, upd_w2, upd_b2, ln_g, ln_b):
    B, N, D = h.shape
    T = 2000
    P = 10240

    H = B // 2
    base = (jnp.arange(B, dtype=jnp.int32) * N)[:, None]
    ext = jnp.concatenate(
        [tour[:, -1:], tour, tour[:, :1],
         jnp.zeros((B, P - N - 2), jnp.int32)], axis=1) + base
    h_flat = h.reshape(B * N, D)

    nch = (H * P) // _CHUNK
    gather = _make_sc_gather(nch, D)

    def half_idx(e):
        g = jnp.concatenate([e.reshape(-1), jnp.zeros((8 * _CHUNK,), jnp.int32)])
        return g.reshape(nch + 8, _CHUNK)

    xp_a = gather(h_flat, half_idx(ext[:H])).reshape(H, P, D)
    xp_b = gather(h_flat, half_idx(ext[H:])).reshape(H, P, D)

    bf16 = jnp.bfloat16
    weights = (
        msg_w1[:D].astype(bf16), msg_w1[D:].astype(bf16), msg_w2.astype(bf16),
        msg_b1.reshape(1, D), msg_b2.reshape(1, D),
        upd_w1[:D].astype(bf16), upd_w1[D:].astype(bf16), upd_w2.astype(bf16),
        upd_b1.reshape(1, D), upd_b2.reshape(1, D),
        ln_g.reshape(1, D), ln_b.reshape(1, D),
    )
    hn_a = _tc_compute(xp_a, weights, H, N, D, T).reshape(H * N, D)
    hn_b = _tc_compute(xp_b, weights, H, N, D, T).reshape(H * N, D)

    ncs = (B * N) // _CHUNK
    flat_tour = (tour + base).reshape(-1)
    k0 = (ncs + _NW - 1) // _NW
    pad_rows = _NW * k0 + 8 - ncs
    sidx = jnp.concatenate([flat_tour, flat_tour[:pad_rows * _CHUNK]])
    h_new_flat = _make_sc_scatter2(ncs, D, H * N)(
        hn_a, hn_b, sidx.reshape(ncs + pad_rows, _CHUNK))
    return h_new_flat.reshape(B, N, D)


# speedup vs baseline: 1.3117x; 1.0015x over previous
import functools

import jax
import jax.numpy as jnp
from jax import lax
from jax.experimental import pallas as pl
from jax.experimental.pallas import tpu as pltpu
from jax.experimental.pallas import tpu_sc as plsc

_NC = 2
_NS = 16
_NW = _NC * _NS
_CHUNK = 128
_NBUF = 4


def _idx_block(idx_hbm, idx_v, w, k0):
    c0 = w * k0
    a = (c0 // 8) * 8
    pltpu.sync_copy(idx_hbm.at[pl.ds(a, idx_v.shape[0])], idx_v)
    return c0, c0 - a


def _make_sc_gather(n_chunks, d):
    assert n_chunks % _NW == 0
    k0 = n_chunks // _NW
    kb = (k0 // 8 + 2) * 8
    mesh = plsc.VectorSubcoreMesh(core_axis_name="c", subcore_axis_name="s")

    @functools.partial(
        pl.kernel,
        mesh=mesh,
        out_type=jax.ShapeDtypeStruct((n_chunks * _CHUNK, d), jnp.float32),
        scratch_types=[
            pltpu.VMEM((kb, _CHUNK), jnp.int32),
            pltpu.VMEM((_NBUF, _CHUNK, d), jnp.float32),
            pltpu.SemaphoreType.DMA,
        ],
    )
    def gather(table_hbm, idx_hbm, out_hbm, idx_v, rows_v, sem):
        w = lax.axis_index("s") * _NC + lax.axis_index("c")
        c0, dlt = _idx_block(idx_hbm, idx_v, w, k0)

        for jj in range(_NBUF - 1):
            pltpu.async_copy(table_hbm.at[idx_v.at[dlt + jj]], rows_v.at[jj], sem)

        def body(j, carry):
            ja = j + _NBUF - 1

            @pl.when(ja < k0)
            def _():
                pltpu.async_copy(
                    table_hbm.at[idx_v.at[dlt + ja]], rows_v.at[ja % _NBUF], sem)

            pltpu.make_async_copy(
                table_hbm.at[idx_v.at[dlt + j]], rows_v.at[j % _NBUF], sem).wait()
            pltpu.sync_copy(rows_v.at[j % _NBUF],
                            out_hbm.at[pl.ds((c0 + j) * _CHUNK, _CHUNK)])
            return carry

        lax.fori_loop(0, k0, body, 0)

    return gather


def _make_sc_scatter2(n_chunks, d, hr):
    k0 = (n_chunks + _NW - 1) // _NW
    kb = (k0 // 8 + 2) * 8
    cb = hr // _CHUNK
    la = hr - cb * _CHUNK
    assert 0 < la < _CHUNK and la % 8 == 0
    wb, jb = cb // k0, cb % k0
    mesh = plsc.VectorSubcoreMesh(core_axis_name="c", subcore_axis_name="s")

    @functools.partial(
        pl.kernel,
        mesh=mesh,
        out_type=jax.ShapeDtypeStruct((n_chunks * _CHUNK, d), jnp.float32),
        scratch_types=[
            pltpu.VMEM((kb, _CHUNK), jnp.int32),
            pltpu.VMEM((_NBUF, _CHUNK, d), jnp.float32),
            pltpu.SemaphoreType.DMA,
        ],
    )
    def scatter(vals_a, vals_b, idx_hbm, out_hbm, idx_v, rows_v, sem):
        w = lax.axis_index("s") * _NC + lax.axis_index("c")
        c0, dlt = _idx_block(idx_hbm, idx_v, w, k0)

        def issue(cc, buf, pred):
            @pl.when((cc < cb) & pred)
            def _():
                pltpu.async_copy(
                    vals_a.at[pl.ds(cc * _CHUNK, _CHUNK)], rows_v.at[buf], sem)

            @pl.when((cc > cb) & (cc < n_chunks) & pred)
            def _():
                pltpu.async_copy(
                    vals_b.at[pl.ds(cc * _CHUNK - hr, _CHUNK)],
                    rows_v.at[buf], sem)

        for jj in range(_NBUF - 1):
            issue(c0 + jj, jj, jj < k0)

        def body(j, carry):
            c = c0 + j
            ja = j + _NBUF - 1
            issue(c0 + ja, ja % _NBUF, ja < k0)

            @pl.when((c < n_chunks) & (c != cb))
            def _():
                pltpu.make_async_copy(
                    vals_a.at[pl.ds(0, _CHUNK)], rows_v.at[j % _NBUF], sem).wait()
                pltpu.sync_copy(rows_v.at[j % _NBUF], out_hbm.at[idx_v.at[dlt + j]])

            return carry

        lax.fori_loop(0, k0, body, 0)

        @pl.when(w == wb)
        def _():
            pltpu.sync_copy(vals_a.at[pl.ds(cb * _CHUNK, la)],
                            rows_v.at[0, pl.ds(0, la)])
            pltpu.sync_copy(vals_b.at[pl.ds(0, _CHUNK - la)],
                            rows_v.at[0, pl.ds(la, _CHUNK - la)])
            pltpu.sync_copy(rows_v.at[0], out_hbm.at[idx_v.at[dlt + jb]])

    return scatter


def _make_tc_body(T, D):
    def body(xp_ref, w1a_ref, w1b_ref, w2_ref, b1_ref, b2_ref,
             u1a_ref, u1b_ref, u2_ref, ub1_ref, ub2_ref,
             g_ref, beta_ref, out_ref):
        t = pl.program_id(1)
        f32 = jnp.float32
        bf16 = jnp.bfloat16

        def silu(x):
            one = jnp.asarray(1.0, x.dtype)
            c = jnp.asarray(-1.4426950408889634, x.dtype)
            return x / (one + jnp.exp2(x * c))

        A = xp_ref[0, pl.ds(t * T, T + 2), :]
        xc = A[1:T + 1]
        xcb = xc.astype(bf16)
        g_h = jnp.dot(A.astype(bf16), w1b_ref[...], preferred_element_type=f32)
        fb = (jnp.dot(xcb, w1a_ref[...], preferred_element_type=f32)
              + b1_ref[0])
        yp = (fb + g_h[0:T]).astype(bf16)
        yn = (fb + g_h[2:T + 2]).astype(bf16)
        s = silu(yp) + silu(yn)
        msg = (jnp.dot(s, w2_ref[...], preferred_element_type=f32)
               + 2.0 * b2_ref[0])
        u = (jnp.dot(xcb, u1a_ref[...], preferred_element_type=f32)
             + jnp.dot(msg.astype(bf16), u1b_ref[...], preferred_element_type=f32)
             + ub1_ref[0])
        u = silu(u.astype(bf16))
        upd = (jnp.dot(u, u2_ref[...], preferred_element_type=f32)
               + ub2_ref[0])
        r = xc + upd
        mu = jnp.mean(r, axis=-1, keepdims=True)
        var = jnp.mean((r - mu) ** 2, axis=-1, keepdims=True)
        out_ref[0] = (r - mu) * lax.rsqrt(var + 1e-5) * g_ref[0] + beta_ref[0]

    return body


def _tc_compute(xp, weights, B, N, D, T):
    NT = N // T
    P = xp.shape[1]
    (w1a, w1b, w2, b1, b2, u1a, u1b, u2, ub1, ub2, g, beta) = weights

    def wspec(shape):
        return pl.BlockSpec(shape, lambda b, t: (0,) * len(shape))

    return pl.pallas_call(
        _make_tc_body(T, D),
        grid=(B, NT),
        in_specs=[
            pl.BlockSpec((1, P, D), lambda b, t: (b, 0, 0)),
            wspec((D, D)), wspec((D, D)), wspec((D, D)),
            wspec((1, D)), wspec((1, D)),
            wspec((D, D)), wspec((D, D)), wspec((D, D)),
            wspec((1, D)), wspec((1, D)),
            wspec((1, D)), wspec((1, D)),
        ],
        out_specs=pl.BlockSpec((1, T, D), lambda b, t: (b, t, 0)),
        out_shape=jax.ShapeDtypeStruct((B, N, D), jnp.float32),
        compiler_params=pltpu.CompilerParams(
            dimension_semantics=("arbitrary", "arbitrary"),
        ),
    )(xp, w1a, w1b, w2, b1, b2, u1a, u1b, u2, ub1, ub2, g, beta)


def kernel(h, tour, msg_w1, msg_b1, msg_w2, msg_b2,
           upd_w1, upd_b1, upd_w2, upd_b2, ln_g, ln_b):
    B, N, D = h.shape
    T = 2000
    P = 10240

    H = B // 2
    base = (jnp.arange(B, dtype=jnp.int32) * N)[:, None]
    ext = jnp.concatenate(
        [tour[:, -1:], tour, tour[:, :1],
         jnp.zeros((B, P - N - 2), jnp.int32)], axis=1) + base
    h_flat = h.reshape(B * N, D)

    nch = (H * P) // _CHUNK
    gather = _make_sc_gather(nch, D)

    def half_idx(e):
        g = jnp.concatenate([e.reshape(-1), jnp.zeros((8 * _CHUNK,), jnp.int32)])
        return g.reshape(nch + 8, _CHUNK)

    xp_a = gather(h_flat, half_idx(ext[:H])).reshape(H, P, D)
    xp_b = gather(h_flat, half_idx(ext[H:])).reshape(H, P, D)

    bf16 = jnp.bfloat16
    weights = (
        msg_w1[:D].astype(bf16), msg_w1[D:].astype(bf16), msg_w2.astype(bf16),
        msg_b1.reshape(1, D), msg_b2.reshape(1, D),
        upd_w1[:D].astype(bf16), upd_w1[D:].astype(bf16), upd_w2.astype(bf16),
        upd_b1.reshape(1, D), upd_b2.reshape(1, D),
        ln_g.reshape(1, D), ln_b.reshape(1, D),
    )
    hn_a = _tc_compute(xp_a, weights, H, N, D, T).reshape(H * N, D)
    hn_b = _tc_compute(xp_b, weights, H, N, D, T).reshape(H * N, D)

    ncs = (B * N) // _CHUNK
    flat_tour = (tour + base).reshape(-1)
    k0 = (ncs + _NW - 1) // _NW
    pad_rows = _NW * k0 + 8 - ncs
    sidx = jnp.concatenate([flat_tour, flat_tour[:pad_rows * _CHUNK]])
    h_new_flat = _make_sc_scatter2(ncs, D, H * N)(
        hn_a, hn_b, sidx.reshape(ncs + pad_rows, _CHUNK))
    return h_new_flat.reshape(B, N, D)
